# baseline (speedup 1.0000x reference)
import functools

import jax
import jax.numpy as jnp
from jax import lax
from jax.experimental import pallas as pl
from jax.experimental.pallas import tpu as pltpu
from jax.experimental.pallas import tpu_sc as plsc

_CT = 128
_BC = 512


@functools.lru_cache(maxsize=None)
def _make_prep(V, D):
    import math

    def body(x_ref, o_ref):
        x = x_ref[...]
        ng = _BC // 128
        x4 = x.reshape(D, ng, D, 2)
        o_ref[...] = x4.transpose((1, 2, 3, 0)).reshape(_BC // 2, 2 * D)

    return pl.pallas_call(
        body,
        grid=(math.ceil(V / _BC),),
        in_specs=[pl.BlockSpec((D, _BC), lambda i: (0, i))],
        out_specs=pl.BlockSpec((_BC // 2, 2 * D), lambda i: (i, 0)),
        out_shape=jax.ShapeDtypeStruct((V // 2, 2 * D), jnp.float32),
    )


@functools.lru_cache(maxsize=None)
def _make_kernel(NT, S, D, V2):
    info = plsc.get_sparse_core_info()
    nc, ns = info.num_cores, info.num_subcores
    nw = nc * ns
    t_per_w = NT // nw
    i_per_w = S * t_per_w
    nq = t_per_w // _CT
    n_chunks = S * nq
    mesh = plsc.VectorSubcoreMesh(core_axis_name="c", subcore_axis_name="s")

    @functools.partial(
        pl.kernel,
        out_type=jax.ShapeDtypeStruct((S * D, NT), jnp.float32),
        mesh=mesh,
        scratch_types=[
            pltpu.VMEM((i_per_w,), jnp.int32),
            pltpu.VMEM((_CT,), jnp.int32),
            pltpu.VMEM((_CT,), jnp.int32),
            pltpu.VMEM((_CT,), jnp.int32),
            pltpu.VMEM((_CT,), jnp.int32),
            pltpu.VMEM((_CT, 2 * D), jnp.float32),
            pltpu.VMEM((_CT, 2 * D), jnp.float32),
            pltpu.VMEM((D, _CT), jnp.float32),
            pltpu.VMEM((D, _CT), jnp.float32),
            pltpu.VMEM((256,), jnp.int32),
            pltpu.VMEM((256,), jnp.int32),
            pltpu.SemaphoreType.DMA,
            pltpu.SemaphoreType.DMA,
        ],
        compiler_params=pltpu.CompilerParams(
            use_tc_tiling_on_sc=True,
            needs_layout_passes=False,
            disable_bounds_checks=True,
        ),
    )
    def gather_kernel(tab2, tids_arr, out_ph, idx_v,
                      sidx0, sidx1, par0, par1, gbuf0, gbuf1, obuf0, obuf1,
                      rot1d, rot128, gsem, osem):
        wid = lax.axis_index("s") * nc + lax.axis_index("c")
        i0 = wid * t_per_w
        pltpu.sync_copy(tids_arr.at[pl.ds(wid * i_per_w, i_per_w)], idx_v)

        lanes = jax.lax.iota(jnp.int32, 16)
        for k in range(16):
            rk = (lanes + k) & 15
            rot1d[pl.ds(k * 16, 16)] = rk
            rot128[pl.ds(k * 16, 16)] = rk << 7

        def compute_sidx(c, sidx_ref, par_ref):
            for j in range(_CT // 16):
                ids = idx_v[pl.ds(c * _CT + j * 16, 16)]
                sidx_ref[pl.ds(j * 16, 16)] = ids >> 1
                par_ref[pl.ds(j * 16, 16)] = (ids & 1) << 6

        def gather_start(sidx_ref, gbuf):
            pltpu.async_copy(tab2.at[sidx_ref], gbuf, gsem)

        def transpose_chunk(gbuf, par_ref, obuf):
            @plsc.parallel_loop(0, _CT // 16, unroll=2)
            def jbody(jb):
                tvec = jb * 16 + lanes
                par_vec = par_ref[pl.ds(jb * 16, 16)]
                for k in range(16):
                    rotk = rot1d[pl.ds(k * 16, 16)]
                    parrot = par_vec + rotk
                    for d0 in range(0, D, 16):
                        v = plsc.load_gather(gbuf, [tvec, parrot + d0])
                        plsc.store_scatter(obuf, [rotk + d0, tvec], v)

        def write_start(c, obuf):
            s1 = c // nq
            q1 = c % nq
            pltpu.async_copy(
                obuf,
                out_ph.at[pl.ds(s1 * D, D), pl.ds(i0 + q1 * _CT, _CT)],
                osem,
            )

        def drain_gather():
            pltpu.make_async_copy(
                tab2.at[pl.ds(0, _CT)], gbuf0, gsem
            ).wait()

        def drain_write():
            pltpu.make_async_copy(
                obuf0, out_ph.at[pl.ds(0, D), pl.ds(0, _CT)],
                osem
            ).wait()

        compute_sidx(0, sidx0, par0)
        gather_start(sidx0, gbuf0)

        def step(c, carry):
            even = c % 2 == 0

            @pl.when(c >= 2)
            def _():
                drain_write()

            @pl.when(jnp.logical_and(c + 1 < n_chunks, even))
            def _():
                compute_sidx(c + 1, sidx1, par1)

            @pl.when(jnp.logical_and(c + 1 < n_chunks, jnp.logical_not(even)))
            def _():
                compute_sidx(c + 1, sidx0, par0)

            drain_gather()

            @pl.when(jnp.logical_and(c + 1 < n_chunks, even))
            def _():
                gather_start(sidx1, gbuf1)

            @pl.when(jnp.logical_and(c + 1 < n_chunks, jnp.logical_not(even)))
            def _():
                gather_start(sidx0, gbuf0)

            @pl.when(even)
            def _():
                transpose_chunk(gbuf0, par0, obuf0)
                write_start(c, obuf0)

            @pl.when(jnp.logical_not(even))
            def _():
                transpose_chunk(gbuf1, par1, obuf1)
                write_start(c, obuf1)

            return carry

        lax.fori_loop(0, n_chunks, step, 0)
        drain_write()
        drain_write()

    return gather_kernel


def kernel(token_ids, embedding):
    V, D = embedding.shape
    NT, S = token_ids.shape
    info = plsc.get_sparse_core_info()
    nw = info.num_cores * info.num_subcores
    t_per_w = NT // nw
    tab2 = _make_prep(V, D)(embedding.T)
    tids_arr = jnp.transpose(
        token_ids.T.reshape(S, nw, t_per_w), (1, 0, 2)
    ).reshape(-1)
    out_ph = _make_kernel(NT, S, D, V // 2)(tab2, tids_arr)
    return out_ph.reshape(S, D, NT).transpose(2, 0, 1)


# speedup vs baseline: 6.0244x; 6.0244x over previous
import functools

import jax
import jax.numpy as jnp
from jax import lax
from jax.experimental import pallas as pl
from jax.experimental.pallas import tpu as pltpu
from jax.experimental.pallas import tpu_sc as plsc

_CT = 128
_BC = 512


@functools.lru_cache(maxsize=None)
def _make_prep(V, D):
    import math

    n_blocks = math.ceil(V / _BC)

    def body(x_ref, o_ref):
        xt = x_ref[...].T
        for g in range(_BC // 128):
            o_ref[pl.ds(g * 64, 64), pl.ds(0, D)] = (
                xt[g * 128:g * 128 + 64, :])
            o_ref[pl.ds(g * 64, 64), pl.ds(D, D)] = (
                xt[g * 128 + 64:g * 128 + 128, :])

    return pl.pallas_call(
        body,
        grid=(n_blocks,),
        in_specs=[pl.BlockSpec((D, _BC), lambda i: (0, i))],
        out_specs=pl.BlockSpec((_BC // 2, 2 * D), lambda i: (i, 0)),
        out_shape=jax.ShapeDtypeStruct((n_blocks * _BC // 2, 2 * D),
                                       jnp.float32),
    )


@functools.lru_cache(maxsize=None)
def _make_kernel(NT, S, D, V2):
    info = plsc.get_sparse_core_info()
    nc, ns = info.num_cores, info.num_subcores
    nw = nc * ns
    t_per_w = NT // nw
    i_per_w = S * t_per_w
    nq = t_per_w // _CT
    n_chunks = S * nq
    mesh = plsc.VectorSubcoreMesh(core_axis_name="c", subcore_axis_name="s")

    @functools.partial(
        pl.kernel,
        out_type=jax.ShapeDtypeStruct((S * D, NT), jnp.float32),
        mesh=mesh,
        scratch_types=[
            pltpu.VMEM((i_per_w,), jnp.int32),
            pltpu.VMEM((_CT,), jnp.int32),
            pltpu.VMEM((_CT,), jnp.int32),
            pltpu.VMEM((_CT,), jnp.int32),
            pltpu.VMEM((_CT,), jnp.int32),
            pltpu.VMEM((_CT, 2 * D), jnp.float32),
            pltpu.VMEM((_CT, 2 * D), jnp.float32),
            pltpu.VMEM((D, _CT), jnp.float32),
            pltpu.VMEM((D, _CT), jnp.float32),
            pltpu.VMEM((256,), jnp.int32),
            pltpu.VMEM((256,), jnp.int32),
            pltpu.SemaphoreType.DMA,
            pltpu.SemaphoreType.DMA,
        ],
        compiler_params=pltpu.CompilerParams(
            use_tc_tiling_on_sc=True,
            needs_layout_passes=False,
            disable_bounds_checks=True,
        ),
    )
    def gather_kernel(tab2, tids_arr, out_ph, idx_v,
                      sidx0, sidx1, par0, par1, gbuf0, gbuf1, obuf0, obuf1,
                      rot1d, rot128, gsem, osem):
        wid = lax.axis_index("s") * nc + lax.axis_index("c")
        i0 = wid * t_per_w
        pltpu.sync_copy(tids_arr.at[pl.ds(wid * i_per_w, i_per_w)], idx_v)

        lanes = jax.lax.iota(jnp.int32, 16)
        for k in range(16):
            rk = (lanes + k) & 15
            rot1d[pl.ds(k * 16, 16)] = rk
            rot128[pl.ds(k * 16, 16)] = rk << 7

        def compute_sidx(c, sidx_ref, par_ref):
            for j in range(_CT // 16):
                ids = idx_v[pl.ds(c * _CT + j * 16, 16)]
                sidx_ref[pl.ds(j * 16, 16)] = ((ids >> 7) << 6) | (ids & 63)
                par_ref[pl.ds(j * 16, 16)] = ids & 64

        def gather_start(sidx_ref, gbuf):
            pltpu.async_copy(tab2.at[sidx_ref], gbuf, gsem)

        def transpose_chunk(gbuf, par_ref, obuf):
            @plsc.parallel_loop(0, _CT // 16, unroll=2)
            def jbody(jb):
                tvec = jb * 16 + lanes
                par_vec = par_ref[pl.ds(jb * 16, 16)]
                for k in range(16):
                    rotk = rot1d[pl.ds(k * 16, 16)]
                    parrot = par_vec + rotk
                    for d0 in range(0, D, 16):
                        v = plsc.load_gather(gbuf, [tvec, parrot + d0])
                        plsc.store_scatter(obuf, [rotk + d0, tvec], v)

        def write_start(c, obuf):
            s1 = c // nq
            q1 = c % nq
            pltpu.async_copy(
                obuf,
                out_ph.at[pl.ds(s1 * D, D), pl.ds(i0 + q1 * _CT, _CT)],
                osem,
            )

        def drain_gather():
            pltpu.make_async_copy(
                tab2.at[pl.ds(0, _CT)], gbuf0, gsem
            ).wait()

        def drain_write():
            pltpu.make_async_copy(
                obuf0, out_ph.at[pl.ds(0, D), pl.ds(0, _CT)],
                osem
            ).wait()

        compute_sidx(0, sidx0, par0)
        gather_start(sidx0, gbuf0)

        def step(c, carry):
            even = c % 2 == 0

            @pl.when(c >= 2)
            def _():
                drain_write()

            @pl.when(jnp.logical_and(c + 1 < n_chunks, even))
            def _():
                compute_sidx(c + 1, sidx1, par1)

            @pl.when(jnp.logical_and(c + 1 < n_chunks, jnp.logical_not(even)))
            def _():
                compute_sidx(c + 1, sidx0, par0)

            drain_gather()

            @pl.when(jnp.logical_and(c + 1 < n_chunks, even))
            def _():
                gather_start(sidx1, gbuf1)

            @pl.when(jnp.logical_and(c + 1 < n_chunks, jnp.logical_not(even)))
            def _():
                gather_start(sidx0, gbuf0)

            @pl.when(even)
            def _():
                transpose_chunk(gbuf0, par0, obuf0)
                write_start(c, obuf0)

            @pl.when(jnp.logical_not(even))
            def _():
                transpose_chunk(gbuf1, par1, obuf1)
                write_start(c, obuf1)

            return carry

        lax.fori_loop(0, n_chunks, step, 0)
        drain_write()
        drain_write()

    return gather_kernel


def kernel(token_ids, embedding):
    V, D = embedding.shape
    NT, S = token_ids.shape
    info = plsc.get_sparse_core_info()
    nw = info.num_cores * info.num_subcores
    t_per_w = NT // nw
    tab2 = _make_prep(V, D)(embedding.T)
    tids_arr = jnp.transpose(
        token_ids.T.reshape(S, nw, t_per_w), (1, 0, 2)
    ).reshape(-1)
    out_ph = _make_kernel(NT, S, D, tab2.shape[0])(tab2, tids_arr)
    return out_ph.reshape(S, D, NT).transpose(2, 0, 1)


# speedup vs baseline: 9.9668x; 1.6544x over previous
import functools

import jax
import jax.numpy as jnp
from jax import lax
from jax.experimental import pallas as pl
from jax.experimental.pallas import tpu as pltpu
from jax.experimental.pallas import tpu_sc as plsc

_CT = 128


@functools.lru_cache(maxsize=None)
def _make_kernel(NT, S, D, V2):
    info = plsc.get_sparse_core_info()
    nc, ns = info.num_cores, info.num_subcores
    nw = nc * ns
    t_per_w = NT // nw
    i_per_w = S * t_per_w
    nq = t_per_w // _CT
    n_chunks = S * nq
    mesh = plsc.VectorSubcoreMesh(core_axis_name="c", subcore_axis_name="s")

    @functools.partial(
        pl.kernel,
        out_type=jax.ShapeDtypeStruct((S * D, NT), jnp.float32),
        mesh=mesh,
        scratch_types=[
            pltpu.VMEM((i_per_w,), jnp.int32),
            pltpu.VMEM((_CT,), jnp.int32),
            pltpu.VMEM((_CT,), jnp.int32),
            pltpu.VMEM((_CT,), jnp.int32),
            pltpu.VMEM((_CT,), jnp.int32),
            pltpu.VMEM((_CT, 2 * D), jnp.float32),
            pltpu.VMEM((_CT, 2 * D), jnp.float32),
            pltpu.VMEM((D, _CT), jnp.float32),
            pltpu.VMEM((D, _CT), jnp.float32),
            pltpu.VMEM((256,), jnp.int32),
            pltpu.VMEM((256,), jnp.int32),
            pltpu.SemaphoreType.DMA,
            pltpu.SemaphoreType.DMA,
        ],
        compiler_params=pltpu.CompilerParams(
            use_tc_tiling_on_sc=True,
            needs_layout_passes=False,
            disable_bounds_checks=True,
        ),
    )
    def gather_kernel(tab2, tids_arr, out_ph, idx_v,
                      sidx0, sidx1, par0, par1, gbuf0, gbuf1, obuf0, obuf1,
                      rot1d, rot128, gsem, osem):
        wid = lax.axis_index("s") * nc + lax.axis_index("c")
        i0 = wid * t_per_w
        pltpu.sync_copy(tids_arr.at[pl.ds(wid * i_per_w, i_per_w)], idx_v)

        lanes = jax.lax.iota(jnp.int32, 16)
        for k in range(16):
            rk = (lanes + k) & 15
            rot1d[pl.ds(k * 16, 16)] = rk
            rot128[pl.ds(k * 16, 16)] = rk << 7

        def compute_sidx(c, sidx_ref, par_ref):
            for j in range(_CT // 16):
                ids = idx_v[pl.ds(c * _CT + j * 16, 16)]
                sidx_ref[pl.ds(j * 16, 16)] = ids >> 1
                par_ref[pl.ds(j * 16, 16)] = (ids & 1) << 6

        def gather_start(sidx_ref, gbuf):
            pltpu.async_copy(tab2.at[sidx_ref], gbuf, gsem)

        def transpose_chunk(gbuf, par_ref, obuf):
            @plsc.parallel_loop(0, _CT // 16, unroll=2)
            def jbody(jb):
                tvec = jb * 16 + lanes
                par_vec = par_ref[pl.ds(jb * 16, 16)]
                for k in range(16):
                    rotk = rot1d[pl.ds(k * 16, 16)]
                    parrot = par_vec + rotk
                    for d0 in range(0, D, 16):
                        v = plsc.load_gather(gbuf, [tvec, parrot + d0])
                        plsc.store_scatter(obuf, [rotk + d0, tvec], v)

        def write_start(c, obuf):
            s1 = c // nq
            q1 = c % nq
            pltpu.async_copy(
                obuf,
                out_ph.at[pl.ds(s1 * D, D), pl.ds(i0 + q1 * _CT, _CT)],
                osem,
            )

        def drain_gather():
            pltpu.make_async_copy(
                tab2.at[pl.ds(0, _CT)], gbuf0, gsem
            ).wait()

        def drain_write():
            pltpu.make_async_copy(
                obuf0, out_ph.at[pl.ds(0, D), pl.ds(0, _CT)],
                osem
            ).wait()

        compute_sidx(0, sidx0, par0)
        gather_start(sidx0, gbuf0)

        def step(c, carry):
            even = c % 2 == 0

            @pl.when(c >= 2)
            def _():
                drain_write()

            @pl.when(jnp.logical_and(c + 1 < n_chunks, even))
            def _():
                compute_sidx(c + 1, sidx1, par1)

            @pl.when(jnp.logical_and(c + 1 < n_chunks, jnp.logical_not(even)))
            def _():
                compute_sidx(c + 1, sidx0, par0)

            drain_gather()

            @pl.when(jnp.logical_and(c + 1 < n_chunks, even))
            def _():
                gather_start(sidx1, gbuf1)

            @pl.when(jnp.logical_and(c + 1 < n_chunks, jnp.logical_not(even)))
            def _():
                gather_start(sidx0, gbuf0)

            @pl.when(even)
            def _():
                transpose_chunk(gbuf0, par0, obuf0)
                write_start(c, obuf0)

            @pl.when(jnp.logical_not(even))
            def _():
                transpose_chunk(gbuf1, par1, obuf1)
                write_start(c, obuf1)

            return carry

        lax.fori_loop(0, n_chunks, step, 0)
        drain_write()
        drain_write()

    return gather_kernel


def kernel(token_ids, embedding):
    V, D = embedding.shape
    NT, S = token_ids.shape
    info = plsc.get_sparse_core_info()
    nw = info.num_cores * info.num_subcores
    t_per_w = NT // nw
    tab2 = embedding.reshape(V // 2, 2 * D)
    tids_arr = jnp.transpose(
        token_ids.T.reshape(S, nw, t_per_w), (1, 0, 2)
    ).reshape(-1)
    out_ph = _make_kernel(NT, S, D, V // 2)(tab2, tids_arr)
    return out_ph.reshape(S, D, NT).transpose(2, 0, 1)
